# baseline (speedup 1.0000x reference)
import functools

import jax
import jax.numpy as jnp
from jax import lax
from jax.experimental import pallas as pl
from jax.experimental.pallas import tpu as pltpu
from jax.experimental.pallas import tpu_sc as plsc

N_NODES = 10000
T = 12
D = 128
E = 320000
K = 3

NUM_SC = 2
NUM_TILES = 16
T_PER_SC = T // NUM_SC
CPT = D // NUM_TILES
LANES = 16
GSIZE = 16

CHUNK = 1280
NCHUNK = E // CHUNK
SUBG = CHUNK // 128
NGRP = CHUNK // GSIZE


_NB1 = 1000


def _k1_body(h_ref, w_ref, out_ref):
    x = h_ref[...].reshape(_NB1 * T, D)
    out_ref[...] = jnp.dot(x, w_ref[...],
                           preferred_element_type=jnp.float32).reshape(_NB1, T, D)


def _hw_dense(H, W):
    grid = (N_NODES // _NB1,)
    return pl.pallas_call(
        _k1_body,
        grid=grid,
        in_specs=[
            pl.BlockSpec((_NB1, T, D), lambda n: (n, 0, 0)),
            pl.BlockSpec((D, D), lambda n: (0, 0)),
        ],
        out_specs=pl.BlockSpec((_NB1, T, D), lambda n: (n, 0, 0)),
        out_shape=jax.ShapeDtypeStruct((N_NODES, T, D), jnp.float32),
    )(H, W)


def _sc_body(hws, colT, row, val, zpl, out,
             colc0, colc1, rowc0, rowc1, valc0, valc1, ix, g0, g1, acc,
             se0, se1, sg0, sg1):
    c = lax.axis_index("c")
    s = lax.axis_index("s")

    esems = (se0, se1)
    gsems = (sg0, sg1)
    colcs = (colc0, colc1)
    rowcs = (rowc0, rowc1)
    valcs = (valc0, valc1)
    gbufs = (g0, g1)
    iota = lax.iota(jnp.int32, LANES)

    def edge_issue(slot, j):
        ofs = j * CHUNK
        pltpu.async_copy(colT.at[pl.ds(ofs, CHUNK)], colcs[slot], esems[slot])
        pltpu.async_copy(row.at[pl.ds(ofs, CHUNK)], rowcs[slot], esems[slot])
        pltpu.async_copy(val.at[pl.ds(ofs, CHUNK)], valcs[slot], esems[slot])

    def edge_wait(slot, j):
        ofs = j * CHUNK
        pltpu.make_async_copy(colT.at[pl.ds(ofs, CHUNK)], colcs[slot],
                              esems[slot]).wait()
        pltpu.make_async_copy(row.at[pl.ds(ofs, CHUNK)], rowcs[slot],
                              esems[slot]).wait()
        pltpu.make_async_copy(val.at[pl.ds(ofs, CHUNK)], valcs[slot],
                              esems[slot]).wait()

    def ix_compute(slot, t):
        base = s * (N_NODES * T) + t

        @plsc.parallel_loop(0, SUBG, unroll=2)
        def _(q):
            for p in range(128 // LANES):
                sl = pl.ds(q * 128 + p * LANES, LANES)
                ix[slot * SUBG + q, pl.ds(p * LANES, LANES)] = (
                    colcs[slot][sl] + base)

    def gather_issue(slot):
        for q in range(SUBG):
            pltpu.async_copy(hws.at[ix.at[slot * SUBG + q]],
                             gbufs[slot].at[pl.ds(q * 128, 128)], gsems[slot])

    def gather_wait(slot):
        for q in range(SUBG):
            pltpu.make_async_copy(hws.at[ix.at[slot * SUBG + q]],
                                  gbufs[slot].at[pl.ds(q * 128, 128)],
                                  gsems[slot]).wait()

    def accumulate(slot):
        gb = gbufs[slot]
        rc = rowcs[slot]
        vc = valcs[slot]

        @plsc.parallel_loop(0, NGRP, unroll=8)
        def _(g):
            sl = pl.ds(g * GSIZE, GSIZE)
            rowv = rc[sl]
            valv = vc[sl]
            ev = iota + g * GSIZE
            for ch in range(CPT):
                cv = jnp.full((LANES,), ch, jnp.int32)
                x = plsc.load_gather(gb, [ev, cv])
                plsc.addupdate_scatter(acc, [rowv, cv], x * valv)

    @pl.loop(0, T_PER_SC)
    def _(tl):
        t = c * T_PER_SC + tl

        pltpu.sync_copy(zpl, acc)

        edge_issue(0, 0)
        edge_wait(0, 0)
        ix_compute(0, t)
        gather_issue(0)
        edge_issue(1, 1)

        @pl.loop(0, NCHUNK, step=2)
        def _(j0):
            for b in range(2):
                j = j0 + b
                nb = 1 - b

                @pl.when(j + 1 < NCHUNK)
                def _():
                    edge_wait(nb, j + 1)
                    ix_compute(nb, t)
                    gather_issue(nb)

                gather_wait(b)
                accumulate(b)

                @pl.when(j + 2 < NCHUNK)
                def _():
                    edge_issue(b, j + 2)

        pltpu.sync_copy(acc, out.at[s, t])


_sc_spmm = functools.partial(
    pl.kernel,
    out_type=jax.ShapeDtypeStruct((NUM_TILES, T, N_NODES, CPT), jnp.float32),
    mesh=plsc.VectorSubcoreMesh(core_axis_name="c", subcore_axis_name="s"),
    compiler_params=pltpu.CompilerParams(needs_layout_passes=False,
                                         use_tc_tiling_on_sc=False),
    scratch_types=[
        pltpu.VMEM((CHUNK,), jnp.int32),
        pltpu.VMEM((CHUNK,), jnp.int32),
        pltpu.VMEM((CHUNK,), jnp.int32),
        pltpu.VMEM((CHUNK,), jnp.int32),
        pltpu.VMEM((CHUNK,), jnp.float32),
        pltpu.VMEM((CHUNK,), jnp.float32),
        pltpu.VMEM((2 * SUBG, 128), jnp.int32),
        pltpu.VMEM((CHUNK, CPT), jnp.float32),
        pltpu.VMEM((CHUNK, CPT), jnp.float32),
        pltpu.VMEM((N_NODES, CPT), jnp.float32),
        pltpu.SemaphoreType.DMA,
        pltpu.SemaphoreType.DMA,
        pltpu.SemaphoreType.DMA,
        pltpu.SemaphoreType.DMA,
    ],
)(_sc_body)


_NB2 = 400


def _k3_body(p_ref, h_ref, b_ref, wf_ref, bf_ref, wg_ref, bg_ref,
             gam_ref, bet_ref, out_ref):
    g = jnp.maximum(p_ref[...] + b_ref[...][None], 0.0)
    x = g.reshape(_NB2 * T, D)

    def conv(w_ref, bias_ref):
        m = [jnp.dot(x, w_ref[k], preferred_element_type=jnp.float32)
             .reshape(_NB2, T, D) for k in range(K)]
        z1 = jnp.zeros((_NB2, 1, D), jnp.float32)
        z2 = jnp.zeros((_NB2, 2, D), jnp.float32)
        return (m[2]
                + jnp.concatenate([z1, m[1][:, :T - 1, :]], axis=1)
                + jnp.concatenate([z2, m[0][:, :T - 2, :]], axis=1)
                + bias_ref[...][None])

    f = conv(wf_ref, bf_ref)
    gg = conv(wg_ref, bg_ref)
    h = jnp.tanh(f) * jax.nn.sigmoid(gg)
    r = h + h_ref[...]
    mu = jnp.mean(r, axis=-1, keepdims=True)
    var = jnp.mean((r - mu) ** 2, axis=-1, keepdims=True)
    out_ref[...] = ((r - mu) * lax.rsqrt(var + 1e-5) * gam_ref[...][None]
                    + bet_ref[...][None])


def _fuse_temporal(P, H, b, Wfk, bf, Wgk, bg, gamma, beta):
    grid = (N_NODES // _NB2,)
    vec = pl.BlockSpec((1, D), lambda n: (0, 0))
    return pl.pallas_call(
        _k3_body,
        grid=grid,
        in_specs=[
            pl.BlockSpec((_NB2, T, D), lambda n: (n, 0, 0)),
            pl.BlockSpec((_NB2, T, D), lambda n: (n, 0, 0)),
            vec,
            pl.BlockSpec((K, D, D), lambda n: (0, 0, 0)),
            vec,
            pl.BlockSpec((K, D, D), lambda n: (0, 0, 0)),
            vec,
            vec,
            vec,
        ],
        out_specs=pl.BlockSpec((_NB2, T, D), lambda n: (n, 0, 0)),
        out_shape=jax.ShapeDtypeStruct((N_NODES, T, D), jnp.float32),
    )(P, H, b.reshape(1, D), Wfk, bf.reshape(1, D), Wgk, bg.reshape(1, D),
      gamma.reshape(1, D), beta.reshape(1, D))


def kernel(H, A_indices, A_values, W, b, Wf, bf, Wg, bg, gamma, beta):
    hw = _hw_dense(H, W)
    hws = (hw.reshape(N_NODES, T, NUM_TILES, CPT)
           .transpose(2, 0, 1, 3)
           .reshape(NUM_TILES * N_NODES * T, CPT))

    row = A_indices[0].astype(jnp.int32)
    colT = A_indices[1].astype(jnp.int32) * T
    val = A_values.astype(jnp.float32)
    zpl = jnp.zeros((N_NODES, CPT), jnp.float32)

    P4 = _sc_spmm(hws, colT, row, val, zpl)
    P = P4.transpose(2, 1, 0, 3).reshape(N_NODES, T, D)

    Wfk = jnp.transpose(Wf, (2, 1, 0))
    Wgk = jnp.transpose(Wg, (2, 1, 0))
    return _fuse_temporal(P, H, b, Wfk, bf, Wgk, bg, gamma, beta)


# speedup vs baseline: 1.1775x; 1.1775x over previous
import functools

import jax
import jax.numpy as jnp
from jax import lax
from jax.experimental import pallas as pl
from jax.experimental.pallas import tpu as pltpu
from jax.experimental.pallas import tpu_sc as plsc

N_NODES = 10000
T = 12
D = 128
E = 320000
K = 3

NUM_SC = 2
NUM_TILES = 16
T_PER_SC = T // NUM_SC
CPT = D // NUM_TILES
LANES = 16
GSIZE = 16

CHUNK = 1280
NCHUNK = E // CHUNK
SUBG = CHUNK // 128
NGRP = CHUNK // GSIZE


_NB1 = 1000


def _k1_body(h_ref, w_ref, out_ref):
    x = h_ref[...].reshape(_NB1 * T, D)
    out_ref[...] = jnp.dot(x, w_ref[...],
                           preferred_element_type=jnp.float32).reshape(_NB1, T, D)


def _hw_dense(H, W):
    grid = (N_NODES // _NB1,)
    return pl.pallas_call(
        _k1_body,
        grid=grid,
        in_specs=[
            pl.BlockSpec((_NB1, T, D), lambda n: (n, 0, 0)),
            pl.BlockSpec((D, D), lambda n: (0, 0)),
        ],
        out_specs=pl.BlockSpec((_NB1, T, D), lambda n: (n, 0, 0)),
        out_shape=jax.ShapeDtypeStruct((N_NODES, T, D), jnp.float32),
    )(H, W)


def _sc_body(hws, colT, row, val, zpl, out,
             colc0, colc1, rowc0, rowc1, valc0, valc1, ix, g0, g1, acc,
             se0, se1, sg0, sg1):
    c = lax.axis_index("c")
    s = lax.axis_index("s")

    esems = (se0, se1)
    gsems = (sg0, sg1)
    colcs = (colc0, colc1)
    rowcs = (rowc0, rowc1)
    valcs = (valc0, valc1)
    gbufs = (g0, g1)
    iota = lax.iota(jnp.int32, LANES)

    def edge_issue(slot, j):
        ofs = j * CHUNK
        pltpu.async_copy(colT.at[pl.ds(ofs, CHUNK)], colcs[slot], esems[slot])
        pltpu.async_copy(row.at[pl.ds(ofs, CHUNK)], rowcs[slot], esems[slot])
        pltpu.async_copy(val.at[pl.ds(ofs, CHUNK)], valcs[slot], esems[slot])

    def edge_wait(slot, j):
        ofs = j * CHUNK
        pltpu.make_async_copy(colT.at[pl.ds(ofs, CHUNK)], colcs[slot],
                              esems[slot]).wait()
        pltpu.make_async_copy(row.at[pl.ds(ofs, CHUNK)], rowcs[slot],
                              esems[slot]).wait()
        pltpu.make_async_copy(val.at[pl.ds(ofs, CHUNK)], valcs[slot],
                              esems[slot]).wait()

    def ix_compute(slot, t):
        base = s * (N_NODES * T) + t

        @plsc.parallel_loop(0, SUBG, unroll=2)
        def _(q):
            for p in range(128 // LANES):
                sl = pl.ds(q * 128 + p * LANES, LANES)
                ix[slot * SUBG + q, pl.ds(p * LANES, LANES)] = (
                    colcs[slot][sl] + base)

    def gather_issue(slot):
        for q in range(SUBG):
            pltpu.async_copy(hws.at[ix.at[slot * SUBG + q]],
                             gbufs[slot].at[pl.ds(q * 128, 128)], gsems[slot])

    def gather_wait(slot):
        for q in range(SUBG):
            pltpu.make_async_copy(hws.at[ix.at[slot * SUBG + q]],
                                  gbufs[slot].at[pl.ds(q * 128, 128)],
                                  gsems[slot]).wait()

    def accumulate(slot):
        gb = gbufs[slot]
        rc = rowcs[slot]
        vc = valcs[slot]

        @plsc.parallel_loop(0, NGRP, unroll=2)
        def _(g):
            sl = pl.ds(g * GSIZE, GSIZE)
            rowv = rc[sl]
            valv = vc[sl]
            ev = iota + g * GSIZE
            for ch in range(CPT):
                cv = jnp.full((LANES,), ch, jnp.int32)
                x = plsc.load_gather(gb, [ev, cv])
                plsc.addupdate_scatter(acc, [rowv, cv], x * valv)

    @pl.loop(0, T_PER_SC)
    def _(tl):
        t = c * T_PER_SC + tl

        pltpu.sync_copy(zpl, acc)

        edge_issue(0, 0)
        edge_wait(0, 0)
        ix_compute(0, t)
        gather_issue(0)
        edge_issue(1, 1)

        @pl.loop(0, NCHUNK, step=2)
        def _(j0):
            for b in range(2):
                j = j0 + b
                nb = 1 - b

                @pl.when(j + 1 < NCHUNK)
                def _():
                    edge_wait(nb, j + 1)
                    ix_compute(nb, t)
                    gather_issue(nb)

                gather_wait(b)
                accumulate(b)

                @pl.when(j + 2 < NCHUNK)
                def _():
                    edge_issue(b, j + 2)

        pltpu.sync_copy(acc, out.at[s, t])


_sc_spmm = functools.partial(
    pl.kernel,
    out_type=jax.ShapeDtypeStruct((NUM_TILES, T, N_NODES, CPT), jnp.float32),
    mesh=plsc.VectorSubcoreMesh(core_axis_name="c", subcore_axis_name="s"),
    compiler_params=pltpu.CompilerParams(needs_layout_passes=False,
                                         use_tc_tiling_on_sc=False),
    scratch_types=[
        pltpu.VMEM((CHUNK,), jnp.int32),
        pltpu.VMEM((CHUNK,), jnp.int32),
        pltpu.VMEM((CHUNK,), jnp.int32),
        pltpu.VMEM((CHUNK,), jnp.int32),
        pltpu.VMEM((CHUNK,), jnp.float32),
        pltpu.VMEM((CHUNK,), jnp.float32),
        pltpu.VMEM((2 * SUBG, 128), jnp.int32),
        pltpu.VMEM((CHUNK, CPT), jnp.float32),
        pltpu.VMEM((CHUNK, CPT), jnp.float32),
        pltpu.VMEM((N_NODES, CPT), jnp.float32),
        pltpu.SemaphoreType.DMA,
        pltpu.SemaphoreType.DMA,
        pltpu.SemaphoreType.DMA,
        pltpu.SemaphoreType.DMA,
    ],
)(_sc_body)


_NB2 = 400


def _k3_body(p_ref, h_ref, b_ref, wf_ref, bf_ref, wg_ref, bg_ref,
             gam_ref, bet_ref, out_ref):
    g = jnp.maximum(p_ref[...] + b_ref[...][None], 0.0)
    x = g.reshape(_NB2 * T, D)

    def conv(w_ref, bias_ref):
        m = [jnp.dot(x, w_ref[k], preferred_element_type=jnp.float32)
             .reshape(_NB2, T, D) for k in range(K)]
        z1 = jnp.zeros((_NB2, 1, D), jnp.float32)
        z2 = jnp.zeros((_NB2, 2, D), jnp.float32)
        return (m[2]
                + jnp.concatenate([z1, m[1][:, :T - 1, :]], axis=1)
                + jnp.concatenate([z2, m[0][:, :T - 2, :]], axis=1)
                + bias_ref[...][None])

    f = conv(wf_ref, bf_ref)
    gg = conv(wg_ref, bg_ref)
    h = jnp.tanh(f) * jax.nn.sigmoid(gg)
    r = h + h_ref[...]
    mu = jnp.mean(r, axis=-1, keepdims=True)
    var = jnp.mean((r - mu) ** 2, axis=-1, keepdims=True)
    out_ref[...] = ((r - mu) * lax.rsqrt(var + 1e-5) * gam_ref[...][None]
                    + bet_ref[...][None])


def _fuse_temporal(P, H, b, Wfk, bf, Wgk, bg, gamma, beta):
    grid = (N_NODES // _NB2,)
    vec = pl.BlockSpec((1, D), lambda n: (0, 0))
    return pl.pallas_call(
        _k3_body,
        grid=grid,
        in_specs=[
            pl.BlockSpec((_NB2, T, D), lambda n: (n, 0, 0)),
            pl.BlockSpec((_NB2, T, D), lambda n: (n, 0, 0)),
            vec,
            pl.BlockSpec((K, D, D), lambda n: (0, 0, 0)),
            vec,
            pl.BlockSpec((K, D, D), lambda n: (0, 0, 0)),
            vec,
            vec,
            vec,
        ],
        out_specs=pl.BlockSpec((_NB2, T, D), lambda n: (n, 0, 0)),
        out_shape=jax.ShapeDtypeStruct((N_NODES, T, D), jnp.float32),
    )(P, H, b.reshape(1, D), Wfk, bf.reshape(1, D), Wgk, bg.reshape(1, D),
      gamma.reshape(1, D), beta.reshape(1, D))


def kernel(H, A_indices, A_values, W, b, Wf, bf, Wg, bg, gamma, beta):
    hw = _hw_dense(H, W)
    hws = (hw.reshape(N_NODES, T, NUM_TILES, CPT)
           .transpose(2, 0, 1, 3)
           .reshape(NUM_TILES * N_NODES * T, CPT))

    row = A_indices[0].astype(jnp.int32)
    colT = A_indices[1].astype(jnp.int32) * T
    val = A_values.astype(jnp.float32)
    zpl = jnp.zeros((N_NODES, CPT), jnp.float32)

    P4 = _sc_spmm(hws, colT, row, val, zpl)
    P = P4.transpose(2, 1, 0, 3).reshape(N_NODES, T, D)

    Wfk = jnp.transpose(Wf, (2, 1, 0))
    Wgk = jnp.transpose(Wg, (2, 1, 0))
    return _fuse_temporal(P, H, b, Wfk, bf, Wgk, bg, gamma, beta)


# speedup vs baseline: 1.2826x; 1.0893x over previous
import functools

import jax
import jax.numpy as jnp
from jax import lax
from jax.experimental import pallas as pl
from jax.experimental.pallas import tpu as pltpu
from jax.experimental.pallas import tpu_sc as plsc

N_NODES = 10000
T = 12
D = 128
E = 320000
K = 3

NUM_SC = 2
NUM_TILES = 16
T_PER_SC = T // NUM_SC
CPT = D // NUM_TILES
LANES = 16
GSIZE = 16

CHUNK = 1280
NCHUNK = E // CHUNK
SUBG = CHUNK // 128
NGRP = CHUNK // GSIZE


_NB1 = 1000


def _k1_body(h_ref, w_ref, out_ref):
    x = h_ref[...].reshape(_NB1 * T, D)
    out_ref[...] = jnp.dot(x, w_ref[...],
                           preferred_element_type=jnp.float32).reshape(_NB1, T, D)


def _hw_dense(H, W):
    grid = (N_NODES // _NB1,)
    return pl.pallas_call(
        _k1_body,
        grid=grid,
        in_specs=[
            pl.BlockSpec((_NB1, T, D), lambda n: (n, 0, 0)),
            pl.BlockSpec((D, D), lambda n: (0, 0)),
        ],
        out_specs=pl.BlockSpec((_NB1, T, D), lambda n: (n, 0, 0)),
        out_shape=jax.ShapeDtypeStruct((N_NODES, T, D), jnp.float32),
    )(H, W)


def _sc_body(hws, colT, row, val, zpl, out,
             colc0, colc1, rowc0, rowc1, valc0, valc1, ix, g0, g1, acc,
             se0, se1, sg0, sg1):
    c = lax.axis_index("c")
    s = lax.axis_index("s")

    esems = (se0, se1)
    gsems = (sg0, sg1)
    colcs = (colc0, colc1)
    rowcs = (rowc0, rowc1)
    valcs = (valc0, valc1)
    gbufs = (g0, g1)
    iota = lax.iota(jnp.int32, LANES)

    def edge_issue(slot, j):
        ofs = j * CHUNK
        pltpu.async_copy(colT.at[pl.ds(ofs, CHUNK)], colcs[slot], esems[slot])
        pltpu.async_copy(row.at[pl.ds(ofs, CHUNK)], rowcs[slot], esems[slot])
        pltpu.async_copy(val.at[pl.ds(ofs, CHUNK)], valcs[slot], esems[slot])

    def edge_wait(slot, j):
        ofs = j * CHUNK
        pltpu.make_async_copy(colT.at[pl.ds(ofs, CHUNK)], colcs[slot],
                              esems[slot]).wait()
        pltpu.make_async_copy(row.at[pl.ds(ofs, CHUNK)], rowcs[slot],
                              esems[slot]).wait()
        pltpu.make_async_copy(val.at[pl.ds(ofs, CHUNK)], valcs[slot],
                              esems[slot]).wait()

    def ix_compute(slot, t):
        base = s * (N_NODES * T) + t

        @plsc.parallel_loop(0, SUBG, unroll=2)
        def _(q):
            for p in range(128 // LANES):
                sl = pl.ds(q * 128 + p * LANES, LANES)
                ix[slot * SUBG + q, pl.ds(p * LANES, LANES)] = (
                    colcs[slot][sl] + base)

    def gather_issue(slot):
        for q in range(SUBG):
            pltpu.async_copy(hws.at[ix.at[slot * SUBG + q]],
                             gbufs[slot].at[pl.ds(q * 128, 128)], gsems[slot])

    def gather_wait(slot):
        for q in range(SUBG):
            pltpu.make_async_copy(hws.at[ix.at[slot * SUBG + q]],
                                  gbufs[slot].at[pl.ds(q * 128, 128)],
                                  gsems[slot]).wait()

    def accumulate(slot):
        gb = gbufs[slot]
        rc = rowcs[slot]
        vc = valcs[slot]

        @plsc.parallel_loop(0, NGRP, unroll=2)
        def _(g):
            sl = pl.ds(g * GSIZE, GSIZE)
            rowv = rc[sl]
            valv = vc[sl]
            ev = iota + g * GSIZE
            for ch in range(CPT):
                cv = jnp.full((LANES,), ch, jnp.int32)
                x = plsc.load_gather(gb, [ev, cv])
                plsc.addupdate_scatter(acc, [rowv, cv], x * valv)

    @pl.loop(0, T_PER_SC)
    def _(tl):
        t = c * T_PER_SC + tl

        pltpu.sync_copy(zpl, acc)

        edge_issue(0, 0)
        edge_wait(0, 0)
        ix_compute(0, t)
        gather_issue(0)
        edge_issue(1, 1)

        @pl.loop(0, NCHUNK, step=2)
        def _(j0):
            for b in range(2):
                j = j0 + b
                nb = 1 - b

                @pl.when(j + 1 < NCHUNK)
                def _():
                    edge_wait(nb, j + 1)
                    ix_compute(nb, t)
                    gather_issue(nb)

                gather_wait(b)
                accumulate(b)

                @pl.when(j + 2 < NCHUNK)
                def _():
                    edge_issue(b, j + 2)

        pltpu.sync_copy(acc, out.at[:, t, pl.ds(s * CPT, CPT)])


_sc_spmm = functools.partial(
    pl.kernel,
    out_type=jax.ShapeDtypeStruct((N_NODES, T, D), jnp.float32),
    mesh=plsc.VectorSubcoreMesh(core_axis_name="c", subcore_axis_name="s"),
    compiler_params=pltpu.CompilerParams(needs_layout_passes=False,
                                         use_tc_tiling_on_sc=False),
    scratch_types=[
        pltpu.VMEM((CHUNK,), jnp.int32),
        pltpu.VMEM((CHUNK,), jnp.int32),
        pltpu.VMEM((CHUNK,), jnp.int32),
        pltpu.VMEM((CHUNK,), jnp.int32),
        pltpu.VMEM((CHUNK,), jnp.float32),
        pltpu.VMEM((CHUNK,), jnp.float32),
        pltpu.VMEM((2 * SUBG, 128), jnp.int32),
        pltpu.VMEM((CHUNK, CPT), jnp.float32),
        pltpu.VMEM((CHUNK, CPT), jnp.float32),
        pltpu.VMEM((N_NODES, CPT), jnp.float32),
        pltpu.SemaphoreType.DMA,
        pltpu.SemaphoreType.DMA,
        pltpu.SemaphoreType.DMA,
        pltpu.SemaphoreType.DMA,
    ],
)(_sc_body)


_NB2 = 400


def _k3_body(p_ref, h_ref, b_ref, wf_ref, bf_ref, wg_ref, bg_ref,
             gam_ref, bet_ref, out_ref):
    g = jnp.maximum(p_ref[...] + b_ref[...][None], 0.0)
    x = g.reshape(_NB2 * T, D)

    def conv(w_ref, bias_ref):
        m = [jnp.dot(x, w_ref[k], preferred_element_type=jnp.float32)
             .reshape(_NB2, T, D) for k in range(K)]
        z1 = jnp.zeros((_NB2, 1, D), jnp.float32)
        z2 = jnp.zeros((_NB2, 2, D), jnp.float32)
        return (m[2]
                + jnp.concatenate([z1, m[1][:, :T - 1, :]], axis=1)
                + jnp.concatenate([z2, m[0][:, :T - 2, :]], axis=1)
                + bias_ref[...][None])

    f = conv(wf_ref, bf_ref)
    gg = conv(wg_ref, bg_ref)
    h = jnp.tanh(f) * jax.nn.sigmoid(gg)
    r = h + h_ref[...]
    mu = jnp.mean(r, axis=-1, keepdims=True)
    var = jnp.mean((r - mu) ** 2, axis=-1, keepdims=True)
    out_ref[...] = ((r - mu) * lax.rsqrt(var + 1e-5) * gam_ref[...][None]
                    + bet_ref[...][None])


def _fuse_temporal(P, H, b, Wfk, bf, Wgk, bg, gamma, beta):
    grid = (N_NODES // _NB2,)
    vec = pl.BlockSpec((1, D), lambda n: (0, 0))
    return pl.pallas_call(
        _k3_body,
        grid=grid,
        in_specs=[
            pl.BlockSpec((_NB2, T, D), lambda n: (n, 0, 0)),
            pl.BlockSpec((_NB2, T, D), lambda n: (n, 0, 0)),
            vec,
            pl.BlockSpec((K, D, D), lambda n: (0, 0, 0)),
            vec,
            pl.BlockSpec((K, D, D), lambda n: (0, 0, 0)),
            vec,
            vec,
            vec,
        ],
        out_specs=pl.BlockSpec((_NB2, T, D), lambda n: (n, 0, 0)),
        out_shape=jax.ShapeDtypeStruct((N_NODES, T, D), jnp.float32),
    )(P, H, b.reshape(1, D), Wfk, bf.reshape(1, D), Wgk, bg.reshape(1, D),
      gamma.reshape(1, D), beta.reshape(1, D))


def kernel(H, A_indices, A_values, W, b, Wf, bf, Wg, bg, gamma, beta):
    hw = _hw_dense(H, W)
    hws = (hw.reshape(N_NODES, T, NUM_TILES, CPT)
           .transpose(2, 0, 1, 3)
           .reshape(NUM_TILES * N_NODES * T, CPT))

    row = A_indices[0].astype(jnp.int32)
    colT = A_indices[1].astype(jnp.int32) * T
    val = A_values.astype(jnp.float32)
    zpl = jnp.zeros((N_NODES, CPT), jnp.float32)

    P = _sc_spmm(hws, colT, row, val, zpl)

    Wfk = jnp.transpose(Wf, (2, 1, 0))
    Wgk = jnp.transpose(Wg, (2, 1, 0))
    return _fuse_temporal(P, H, b, Wfk, bf, Wgk, bg, gamma, beta)


# speedup vs baseline: 1.8298x; 1.4266x over previous
import functools

import jax
import jax.numpy as jnp
from jax import lax
from jax.experimental import pallas as pl
from jax.experimental.pallas import tpu as pltpu
from jax.experimental.pallas import tpu_sc as plsc

N_NODES = 10000
T = 12
D = 128
E = 320000
K = 3

NUM_SC = 2
NUM_TILES = 16
T_PER_SC = T // NUM_SC
CPT = D // NUM_TILES
LANES = 16
GSIZE = 16

CHUNK = 1280
NCHUNK = E // CHUNK
SUBG = CHUNK // 128
NGRP = CHUNK // GSIZE


_NB1 = 1000


def _k1_body(h_ref, w_ref, out_ref):
    x = h_ref[...].reshape(_NB1 * T, D)
    out_ref[...] = jnp.dot(x, w_ref[...],
                           preferred_element_type=jnp.float32).reshape(_NB1, T, D)


def _hw_dense(H, W):
    grid = (N_NODES // _NB1,)
    return pl.pallas_call(
        _k1_body,
        grid=grid,
        in_specs=[
            pl.BlockSpec((_NB1, T, D), lambda n: (n, 0, 0)),
            pl.BlockSpec((D, D), lambda n: (0, 0)),
        ],
        out_specs=pl.BlockSpec((_NB1, T, D), lambda n: (n, 0, 0)),
        out_shape=jax.ShapeDtypeStruct((N_NODES, T, D), jnp.float32),
    )(H, W)


NT = N_NODES * T
BCH = 1250
NBCH = NT // BCH


def _sc_body(hws, colT, row, val, zpl, out, table,
             colc0, colc1, rowc0, rowc1, valc0, valc1, ix, g0, g1, acc,
             se0, se1, sg0, sg1):
    c = lax.axis_index("c")
    s = lax.axis_index("s")

    esems = (se0, se1)
    gsems = (sg0, sg1)
    colcs = (colc0, colc1)
    rowcs = (rowc0, rowc1)
    valcs = (valc0, valc1)
    gbufs = (g0, g1)
    iota = lax.iota(jnp.int32, LANES)

    def edge_issue(slot, j):
        ofs = j * CHUNK
        pltpu.async_copy(colT.at[pl.ds(ofs, CHUNK)], colcs[slot], esems[slot])
        pltpu.async_copy(row.at[pl.ds(ofs, CHUNK)], rowcs[slot], esems[slot])
        pltpu.async_copy(val.at[pl.ds(ofs, CHUNK)], valcs[slot], esems[slot])

    def edge_wait(slot, j):
        ofs = j * CHUNK
        pltpu.make_async_copy(colT.at[pl.ds(ofs, CHUNK)], colcs[slot],
                              esems[slot]).wait()
        pltpu.make_async_copy(row.at[pl.ds(ofs, CHUNK)], rowcs[slot],
                              esems[slot]).wait()
        pltpu.make_async_copy(val.at[pl.ds(ofs, CHUNK)], valcs[slot],
                              esems[slot]).wait()

    sec = (c * NUM_TILES + s) * NT

    def build_read(slot, j):
        pltpu.async_copy(
            hws.at[pl.ds(j * BCH, BCH), pl.ds(s * CPT, CPT)],
            gbufs[slot].at[pl.ds(0, BCH)], esems[slot])

    def build_read_wait(slot, j):
        pltpu.make_async_copy(
            hws.at[pl.ds(j * BCH, BCH), pl.ds(s * CPT, CPT)],
            gbufs[slot].at[pl.ds(0, BCH)], esems[slot]).wait()

    build_read(0, 0)

    @pl.loop(0, NBCH, step=2)
    def _(j0):
        for b in range(2):
            j = j0 + b

            build_read_wait(b, j)

            @pl.when(j + 1 < NBCH)
            def _():
                build_read(1 - b, j + 1)

            pltpu.sync_copy(gbufs[b].at[pl.ds(0, BCH)],
                            table.at[pl.ds(sec + j * BCH, BCH)])

    def ix_compute(slot, t):
        base = sec + t

        @plsc.parallel_loop(0, SUBG, unroll=2)
        def _(q):
            for p in range(128 // LANES):
                sl = pl.ds(q * 128 + p * LANES, LANES)
                ix[slot * SUBG + q, pl.ds(p * LANES, LANES)] = (
                    colcs[slot][sl] + base)

    def gather_issue(slot):
        for q in range(SUBG):
            pltpu.async_copy(table.at[ix.at[slot * SUBG + q]],
                             gbufs[slot].at[pl.ds(q * 128, 128)], gsems[slot])

    def gather_wait(slot):
        for q in range(SUBG):
            pltpu.make_async_copy(table.at[ix.at[slot * SUBG + q]],
                                  gbufs[slot].at[pl.ds(q * 128, 128)],
                                  gsems[slot]).wait()

    def accumulate(slot):
        gb = gbufs[slot]
        rc = rowcs[slot]
        vc = valcs[slot]

        @plsc.parallel_loop(0, NGRP, unroll=2)
        def _(g):
            sl = pl.ds(g * GSIZE, GSIZE)
            rowv = rc[sl]
            valv = vc[sl]
            ev = iota + g * GSIZE
            for ch in range(CPT):
                cv = jnp.full((LANES,), ch, jnp.int32)
                x = plsc.load_gather(gb, [ev, cv])
                plsc.addupdate_scatter(acc, [rowv, cv], x * valv)

    @pl.loop(0, T_PER_SC)
    def _(tl):
        t = c * T_PER_SC + tl

        pltpu.sync_copy(zpl, acc)

        edge_issue(0, 0)
        edge_wait(0, 0)
        ix_compute(0, t)
        gather_issue(0)
        edge_issue(1, 1)

        @pl.loop(0, NCHUNK, step=2)
        def _(j0):
            for b in range(2):
                j = j0 + b
                nb = 1 - b

                @pl.when(j + 1 < NCHUNK)
                def _():
                    edge_wait(nb, j + 1)
                    ix_compute(nb, t)
                    gather_issue(nb)

                gather_wait(b)
                accumulate(b)

                @pl.when(j + 2 < NCHUNK)
                def _():
                    edge_issue(b, j + 2)

        pltpu.sync_copy(acc, out.at[:, t, pl.ds(s * CPT, CPT)])


_sc_spmm = functools.partial(
    pl.kernel,
    out_type=(jax.ShapeDtypeStruct((N_NODES, T, D), jnp.float32),
              jax.ShapeDtypeStruct((NUM_SC * NUM_TILES * NT, CPT),
                                   jnp.float32)),
    mesh=plsc.VectorSubcoreMesh(core_axis_name="c", subcore_axis_name="s"),
    compiler_params=pltpu.CompilerParams(needs_layout_passes=False,
                                         use_tc_tiling_on_sc=False),
    scratch_types=[
        pltpu.VMEM((CHUNK,), jnp.int32),
        pltpu.VMEM((CHUNK,), jnp.int32),
        pltpu.VMEM((CHUNK,), jnp.int32),
        pltpu.VMEM((CHUNK,), jnp.int32),
        pltpu.VMEM((CHUNK,), jnp.float32),
        pltpu.VMEM((CHUNK,), jnp.float32),
        pltpu.VMEM((2 * SUBG, 128), jnp.int32),
        pltpu.VMEM((CHUNK, CPT), jnp.float32),
        pltpu.VMEM((CHUNK, CPT), jnp.float32),
        pltpu.VMEM((N_NODES, CPT), jnp.float32),
        pltpu.SemaphoreType.DMA,
        pltpu.SemaphoreType.DMA,
        pltpu.SemaphoreType.DMA,
        pltpu.SemaphoreType.DMA,
    ],
)(_sc_body)


_NB2 = 400


def _k3_body(p_ref, h_ref, b_ref, wf_ref, bf_ref, wg_ref, bg_ref,
             gam_ref, bet_ref, out_ref):
    g = jnp.maximum(p_ref[...] + b_ref[...][None], 0.0)
    x = g.reshape(_NB2 * T, D)

    def conv(w_ref, bias_ref):
        m = [jnp.dot(x, w_ref[k], preferred_element_type=jnp.float32)
             .reshape(_NB2, T, D) for k in range(K)]
        z1 = jnp.zeros((_NB2, 1, D), jnp.float32)
        z2 = jnp.zeros((_NB2, 2, D), jnp.float32)
        return (m[2]
                + jnp.concatenate([z1, m[1][:, :T - 1, :]], axis=1)
                + jnp.concatenate([z2, m[0][:, :T - 2, :]], axis=1)
                + bias_ref[...][None])

    f = conv(wf_ref, bf_ref)
    gg = conv(wg_ref, bg_ref)
    h = jnp.tanh(f) * jax.nn.sigmoid(gg)
    r = h + h_ref[...]
    mu = jnp.mean(r, axis=-1, keepdims=True)
    var = jnp.mean((r - mu) ** 2, axis=-1, keepdims=True)
    out_ref[...] = ((r - mu) * lax.rsqrt(var + 1e-5) * gam_ref[...][None]
                    + bet_ref[...][None])


def _fuse_temporal(P, H, b, Wfk, bf, Wgk, bg, gamma, beta):
    grid = (N_NODES // _NB2,)
    vec = pl.BlockSpec((1, D), lambda n: (0, 0))
    return pl.pallas_call(
        _k3_body,
        grid=grid,
        in_specs=[
            pl.BlockSpec((_NB2, T, D), lambda n: (n, 0, 0)),
            pl.BlockSpec((_NB2, T, D), lambda n: (n, 0, 0)),
            vec,
            pl.BlockSpec((K, D, D), lambda n: (0, 0, 0)),
            vec,
            pl.BlockSpec((K, D, D), lambda n: (0, 0, 0)),
            vec,
            vec,
            vec,
        ],
        out_specs=pl.BlockSpec((_NB2, T, D), lambda n: (n, 0, 0)),
        out_shape=jax.ShapeDtypeStruct((N_NODES, T, D), jnp.float32),
    )(P, H, b.reshape(1, D), Wfk, bf.reshape(1, D), Wgk, bg.reshape(1, D),
      gamma.reshape(1, D), beta.reshape(1, D))


def kernel(H, A_indices, A_values, W, b, Wf, bf, Wg, bg, gamma, beta):
    hws = _hw_dense(H, W).reshape(NT, D)

    row = A_indices[0].astype(jnp.int32)
    colT = A_indices[1].astype(jnp.int32) * T
    val = A_values.astype(jnp.float32)
    zpl = jnp.zeros((N_NODES, CPT), jnp.float32)

    P, _ = _sc_spmm(hws, colT, row, val, zpl)

    Wfk = jnp.transpose(Wf, (2, 1, 0))
    Wgk = jnp.transpose(Wg, (2, 1, 0))
    return _fuse_temporal(P, H, b, Wfk, bf, Wgk, bg, gamma, beta)
